# baseline (speedup 1.0000x reference)
import jax
import jax.numpy as jnp
from jax import lax
from jax.experimental import pallas as pl
from jax.experimental.pallas import tpu as pltpu
from jax.experimental.pallas import tpu_sc as plsc

_B = 4096
_EMB = 64
_ROWS = 100000
_ROWS2 = _ROWS // 2
_NC, _NS = 2, 16
_NW = _NC * _NS
_BPW = _B // _NW

_TBLK = 1024
_TGRID = 49
_SPLIT = _TGRID * _TBLK


def _transpose_body(lo_ref, hi_ref, out_ref):
    ii = lax.broadcasted_iota(jnp.int32, (_EMB, _EMB), 0)
    ident = (ii == ii.T).astype(jnp.bfloat16)
    dim0 = (((0,), (0,)), ((), ()))
    out_ref[:, pl.ds(0, _EMB)] = lax.dot_general(
        lo_ref[...].astype(jnp.bfloat16), ident, dim0,
        preferred_element_type=jnp.float32)
    out_ref[:, pl.ds(_EMB, _EMB)] = lax.dot_general(
        hi_ref[...].astype(jnp.bfloat16), ident, dim0,
        preferred_element_type=jnp.float32)


def _transpose_table(tableT):
    return pl.pallas_call(
        _transpose_body,
        grid=(_TGRID,),
        in_specs=[pl.BlockSpec((_EMB, _TBLK), lambda i: (0, i)),
                  pl.BlockSpec((_EMB, _TBLK), lambda i: (0, i + _TGRID))],
        out_specs=pl.BlockSpec((_TBLK, 2 * _EMB), lambda i: (i, 0)),
        out_shape=jax.ShapeDtypeStruct((_SPLIT, 2 * _EMB), jnp.float32),
    )(tableT, tableT)


def _sc_gather_body(table_hbm, idx_hbm, out_hbm, rows_v, idx_v, sem):
    wid = lax.axis_index("s") * _NC + lax.axis_index("c")
    base = wid * _BPW
    pltpu.sync_copy(idx_hbm.at[pl.ds(base, _BPW)], idx_v)

    def issue(g, carry):
        v = idx_v[pl.ds(g * 16, 16)]
        for k in range(16):
            pltpu.async_copy(table_hbm.at[pl.ds(v[k], 1)],
                             rows_v.at[pl.ds(g * 16 + k, 1)], sem)
        return carry

    lax.fori_loop(0, _BPW // 16, issue, 0)
    pltpu.make_async_copy(table_hbm.at[pl.ds(0, _BPW)], rows_v, sem).wait()
    pltpu.sync_copy(rows_v, out_hbm.at[pl.ds(base, _BPW)])


def _sc_gather(table2, idx):
    return pl.kernel(
        _sc_gather_body,
        mesh=plsc.VectorSubcoreMesh(core_axis_name="c", subcore_axis_name="s"),
        out_type=jax.ShapeDtypeStruct((_B, 2 * _EMB), jnp.float32),
        scratch_types=[
            pltpu.VMEM((_BPW, 2 * _EMB), jnp.float32),
            pltpu.VMEM((_BPW,), jnp.int32),
            pltpu.SemaphoreType.DMA,
        ],
    )(table2, idx)


def _mlp_body(id_emb2, par, sec, stg, reg, numT, sec_t, stg_t, reg_t,
              w1, b1, g1, be1, w2, b2, g2, be2, out):
    f32 = jnp.float32
    bf16 = jnp.bfloat16
    dim0 = (((0,), (0,)), ((), ()))

    def embT(idx_ref, n, table_ref):
        iota = lax.broadcasted_iota(jnp.int32, (n, _B), 0)
        onehot = (iota == idx_ref[...][None, :]).astype(f32)
        return lax.dot_general(onehot, table_ref[...], dim0,
                               preferred_element_type=f32)

    keep = (par[...][:, None] == 0)
    id_emb = jnp.where(keep, id_emb2[:, pl.ds(0, _EMB)],
                       id_emb2[:, pl.ds(_EMB, _EMB)])

    h = (jnp.dot(id_emb.astype(bf16), w1[pl.ds(0, _EMB), :].astype(bf16),
                 preferred_element_type=f32)
         + jnp.dot(embT(sec, 32, sec_t), w1[pl.ds(_EMB, 16), :],
                   preferred_element_type=f32)
         + jnp.dot(embT(stg, 16, stg_t), w1[pl.ds(_EMB + 16, 16), :],
                   preferred_element_type=f32)
         + jnp.dot(embT(reg, 24, reg_t), w1[pl.ds(_EMB + 32, 16), :],
                   preferred_element_type=f32)
         + lax.dot_general(numT[...], w1[pl.ds(_EMB + 48, 6), :], dim0,
                           preferred_element_type=f32)
         + b1[...][None, :])
    h = jnp.maximum(h, 0.0)
    mu = jnp.mean(h, axis=0, keepdims=True)
    var = jnp.mean((h - mu) ** 2, axis=0, keepdims=True)
    h = g1[...][None, :] * (h - mu) * lax.rsqrt(var + 1e-5) + be1[...][None, :]

    h2 = (jnp.dot(h.astype(bf16), w2[...].astype(bf16),
                  preferred_element_type=f32) + b2[...][None, :])
    h2 = jnp.maximum(h2, 0.0)
    mu2 = jnp.mean(h2, axis=0, keepdims=True)
    var2 = jnp.mean((h2 - mu2) ** 2, axis=0, keepdims=True)
    h2 = (g2[...][None, :] * (h2 - mu2) * lax.rsqrt(var2 + 1e-5)
          + be2[...][None, :])

    nrm = jnp.sqrt(jnp.sum(h2 * h2, axis=1, keepdims=True))
    out[...] = h2 / jnp.maximum(nrm, 1e-12)


def kernel(id, sector, stage, region, deal_size, revenue_multiple,
           growth_rate, profitability, team_experience, market_size,
           deal_table, sector_table, stage_table, region_table,
           W1, b1, g1, beta1, W2, b2, g2, beta2):
    idx = id.astype(jnp.int32)
    hi = (idx >= _SPLIT).astype(jnp.int32)
    id_emb2 = _sc_gather(_transpose_table(deal_table.T), idx - hi * _SPLIT)
    numT = jnp.stack([deal_size, revenue_multiple, growth_rate, profitability,
                      team_experience, market_size], axis=0)
    return pl.pallas_call(
        _mlp_body,
        out_shape=jax.ShapeDtypeStruct((_B, W2.shape[1]), jnp.float32),
    )(id_emb2, hi, sector.astype(jnp.int32), stage.astype(jnp.int32),
      region.astype(jnp.int32), numT, sector_table, stage_table, region_table,
      W1, b1, g1, beta1, W2, b2, g2, beta2)


# speedup vs baseline: 1.2043x; 1.2043x over previous
import jax
import jax.numpy as jnp
from jax import lax
from jax.experimental import pallas as pl
from jax.experimental.pallas import tpu as pltpu
from jax.experimental.pallas import tpu_sc as plsc

_B = 4096
_EMB = 64
_NC, _NS = 2, 16
_NW = _NC * _NS
_BPW = _B // _NW


def _sc_gather_body(table_hbm, idx_hbm, out_hbm, rows_v, idx_v, sem):
    wid = lax.axis_index("s") * _NC + lax.axis_index("c")
    base = wid * _BPW
    pltpu.sync_copy(idx_hbm.at[pl.ds(base, _BPW)], idx_v)

    def issue(g, carry):
        v = idx_v[pl.ds(g * 16, 16)]
        for k in range(16):
            pltpu.async_copy(table_hbm.at[pl.ds(v[k], 1)],
                             rows_v.at[pl.ds(g * 16 + k, 1)], sem)
        return carry

    lax.fori_loop(0, _BPW // 16, issue, 0)
    pltpu.make_async_copy(table_hbm.at[pl.ds(0, _BPW)], rows_v, sem).wait()
    pltpu.sync_copy(rows_v, out_hbm.at[pl.ds(base, _BPW)])


def _sc_gather(table, idx):
    return pl.kernel(
        _sc_gather_body,
        mesh=plsc.VectorSubcoreMesh(core_axis_name="c", subcore_axis_name="s"),
        out_type=jax.ShapeDtypeStruct((_B, _EMB), jnp.float32),
        scratch_types=[
            pltpu.VMEM((_BPW, _EMB), jnp.float32),
            pltpu.VMEM((_BPW,), jnp.int32),
            pltpu.SemaphoreType.DMA,
        ],
    )(table, idx)


def _mlp_body(id_emb, sec, stg, reg, numT, sec_tT, stg_t, reg_tT,
              w1, b1, g1, be1, w2, b2, g2, be2, out):
    f32 = jnp.float32
    dim0 = (((0,), (0,)), ((), ()))

    def onehotT(idx_ref, n):
        iota = lax.broadcasted_iota(jnp.int32, (n, _B), 0)
        return (iota == idx_ref[...][None, :]).astype(f32)

    feats = jnp.concatenate(
        [onehotT(sec, 32), onehotT(stg, 16), onehotT(reg, 24), numT[...]],
        axis=0)
    g_small = jnp.concatenate(
        [lax.dot_general(sec_tT[...], w1[pl.ds(_EMB, 16), :], dim0,
                         preferred_element_type=f32),
         jnp.dot(stg_t[...], w1[pl.ds(_EMB + 16, 16), :],
                 preferred_element_type=f32),
         lax.dot_general(reg_tT[...], w1[pl.ds(_EMB + 32, 16), :], dim0,
                         preferred_element_type=f32),
         w1[pl.ds(_EMB + 48, 6), :]],
        axis=0)

    h = (jnp.dot(id_emb[...].astype(jnp.bfloat16),
                 w1[pl.ds(0, _EMB), :].astype(jnp.bfloat16),
                 preferred_element_type=f32)
         + lax.dot_general(feats, g_small, dim0, preferred_element_type=f32)
         + b1[...][None, :])
    h = jnp.maximum(h, 0.0)
    mu = jnp.mean(h, axis=0, keepdims=True)
    var = jnp.mean((h - mu) ** 2, axis=0, keepdims=True)
    h = g1[...][None, :] * (h - mu) * lax.rsqrt(var + 1e-5) + be1[...][None, :]

    h2 = (jnp.dot(h.astype(jnp.bfloat16), w2[...].astype(jnp.bfloat16),
                  preferred_element_type=f32) + b2[...][None, :])
    h2 = jnp.maximum(h2, 0.0)
    mu2 = jnp.mean(h2, axis=0, keepdims=True)
    var2 = jnp.mean((h2 - mu2) ** 2, axis=0, keepdims=True)
    h2 = (g2[...][None, :] * (h2 - mu2) * lax.rsqrt(var2 + 1e-5)
          + be2[...][None, :])

    nrm = jnp.sqrt(jnp.sum(h2 * h2, axis=1, keepdims=True))
    out[...] = h2 / jnp.maximum(nrm, 1e-12)


def kernel(id, sector, stage, region, deal_size, revenue_multiple,
           growth_rate, profitability, team_experience, market_size,
           deal_table, sector_table, stage_table, region_table,
           W1, b1, g1, beta1, W2, b2, g2, beta2):
    id_emb = _sc_gather(deal_table, id.astype(jnp.int32))
    numT = jnp.stack([deal_size, revenue_multiple, growth_rate, profitability,
                      team_experience, market_size], axis=0)
    return pl.pallas_call(
        _mlp_body,
        out_shape=jax.ShapeDtypeStruct((_B, W2.shape[1]), jnp.float32),
    )(id_emb, sector.astype(jnp.int32), stage.astype(jnp.int32),
      region.astype(jnp.int32), numT, sector_table.T, stage_table,
      region_table.T, W1, b1, g1, beta1, W2, b2, g2, beta2)
